# baseline (speedup 1.0000x reference)
import functools

import jax
import jax.numpy as jnp
from jax import lax
from jax.experimental import pallas as pl
from jax.experimental.pallas import tpu as pltpu
from jax.experimental.pallas import tpu_sc as plsc

N = 10000
E = 320000
D = 128

NC = 2
NS = 16
NW = NC * NS
K = 128
BPW = 80
NBLK = BPW * NW
EP = NBLK * K
BPW0 = 96
BPW1 = 2 * BPW - BPW0
NPAD = 10240
SLAB = NPAD // NS


def _sc_mesh():
    return plsc.VectorSubcoreMesh(core_axis_name="c", subcore_axis_name="s")


def _deg_body(coli_hbm, ones_hbm, zeros_hbm, out, cidx_v, cidx_cur, ones_v,
              acc):
    cid = lax.axis_index("c")
    sid = lax.axis_index("s")
    wid = cid * NS + sid
    pltpu.sync_copy(coli_hbm.at[pl.ds(wid * BPW, BPW)], cidx_v)
    pltpu.sync_copy(ones_hbm, ones_v)
    pltpu.sync_copy(zeros_hbm.at[pl.ds(sid * SLAB, SLAB)],
                    acc.at[pl.ds(sid * SLAB, SLAB)])
    plsc.subcore_barrier()

    def step(i, carry):
        for j in range(K // 16):
            cidx_cur[pl.ds(j * 16, 16)] = cidx_v[i, pl.ds(j * 16, 16)]
        pltpu.sync_copy(ones_v, acc.at[cidx_cur], add=True)
        return carry

    lax.fori_loop(0, BPW, step, 0)
    plsc.subcore_barrier()
    pltpu.sync_copy(acc.at[pl.ds(sid * SLAB, SLAB)],
                    out.at[cid, pl.ds(sid * SLAB, SLAB)])


def _deg_call(coli, ones128, zeros128):
    f = functools.partial(
        pl.kernel,
        out_type=jax.ShapeDtypeStruct((NC, NPAD, D), jnp.float32),
        mesh=_sc_mesh(),
        scratch_types=[
            pltpu.VMEM((BPW, K), jnp.int32),
            pltpu.VMEM((K,), jnp.int32),
            pltpu.VMEM((K, D), jnp.float32),
            pltpu.VMEM_SHARED((NPAD, D), jnp.float32),
        ],
    )(_deg_body)
    return f(coli, ones128, zeros128)


NBUF = 4
H = K // 2
CH = 16


def _agg_body(z_hbm, rowi_hbm, coli_hbm, zeros_hbm, out,
              ridx_c, cidx_c, curs, rows, acc, gsems):
    cid = lax.axis_index("c")
    sid = lax.axis_index("s")
    base = jnp.where(cid == 0, sid * BPW0, NS * BPW0 + sid * BPW1)
    nchunks = jnp.where(cid == 0, BPW0 // CH, BPW1 // CH)
    pltpu.sync_copy(zeros_hbm.at[pl.ds(sid * SLAB, SLAB)],
                    acc.at[pl.ds(sid * SLAB, SLAB)])
    plsc.subcore_barrier()

    SB = 2 * CH

    def fire(s, b):
        pltpu.async_copy(
            z_hbm.at[ridx_c.at[s // 2, pl.ds((s % 2) * H, H)]],
            rows[b], gsems[b])

    def drain(b):
        pltpu.make_async_copy(z_hbm.at[ridx_c.at[0, pl.ds(0, H)]],
                              rows[b], gsems[b]).wait()

    def stage(s, cur):
        for j in range(H // 16):
            cur[pl.ds(j * 16, 16)] = cidx_c[s // 2,
                                            pl.ds((s % 2) * H + j * 16, 16)]

    def chunk(q, carry):
        pltpu.sync_copy(rowi_hbm.at[pl.ds(base + q * CH, CH)], ridx_c)
        pltpu.sync_copy(coli_hbm.at[pl.ds(base + q * CH, CH)], cidx_c)
        for s in range(NBUF - 1):
            fire(s, s)
        for s in range(SB):
            b = s % NBUF
            if s + NBUF - 1 < SB:
                fire(s + NBUF - 1, (s + NBUF - 1) % NBUF)
            stage(s, curs[b])
            drain(b)
            pltpu.sync_copy(rows[b], acc.at[curs[b]], add=True)
        return carry

    lax.fori_loop(0, nchunks, chunk, 0)
    plsc.subcore_barrier()
    pltpu.sync_copy(acc.at[pl.ds(sid * SLAB, SLAB)],
                    out.at[cid, pl.ds(sid * SLAB, SLAB)])


def _agg_call(z, rowi, coli, zeros128):
    f = functools.partial(
        pl.kernel,
        out_type=jax.ShapeDtypeStruct((NC, NPAD, D), jnp.float32),
        mesh=_sc_mesh(),
        scratch_types=[
            pltpu.VMEM((CH, K), jnp.int32),
            pltpu.VMEM((CH, K), jnp.int32),
            [pltpu.VMEM((H,), jnp.int32) for _ in range(NBUF)],
            [pltpu.VMEM((H, D), jnp.float32) for _ in range(NBUF)],
            pltpu.VMEM_SHARED((NPAD, D), jnp.float32),
            [pltpu.SemaphoreType.DMA for _ in range(NBUF)],
        ],
    )(_agg_body)
    return f(z, rowi, coli, zeros128)


def _prep_body(xs_ref, w_ref, d0_ref, d1_ref, z_ref, s_ref):
    deg = d0_ref[0, :, 0:1] + d1_ref[0, :, 0:1] + 1.0
    dinv = lax.rsqrt(deg)
    xw = lax.dot_general(xs_ref[...], w_ref[...], (((1,), (1,)), ((), ())),
                         preferred_element_type=jnp.float32)
    z = dinv * xw
    z_ref[...] = z
    s_ref[...] = dinv * z


def _prep_call(xs, W, dp):
    RB = 1000
    return pl.pallas_call(
        _prep_body,
        grid=(N // RB,),
        in_specs=[
            pl.BlockSpec((RB, D), lambda i: (i, 0)),
            pl.BlockSpec((D, D), lambda i: (0, 0)),
            pl.BlockSpec((1, RB, D), lambda i: (0, i, 0)),
            pl.BlockSpec((1, RB, D), lambda i: (1, i, 0)),
        ],
        out_specs=[
            pl.BlockSpec((RB, D), lambda i: (i, 0)),
            pl.BlockSpec((RB, D), lambda i: (i, 0)),
        ],
        out_shape=[
            jax.ShapeDtypeStruct((N, D), jnp.float32),
            jax.ShapeDtypeStruct((N, D), jnp.float32),
        ],
    )(xs, W, dp, dp)


def _fin_body(a0_ref, a1_ref, s_ref, d0_ref, d1_ref, b_ref, g_ref, be_ref,
              y_ref):
    deg = d0_ref[0, :, 0:1] + d1_ref[0, :, 0:1] + 1.0
    dinv = lax.rsqrt(deg)
    out = (dinv * (a0_ref[0] + a1_ref[0]) + s_ref[...] + b_ref[...])
    mu = jnp.mean(out, axis=1, keepdims=True)
    dev = out - mu
    var = jnp.mean(dev * dev, axis=1, keepdims=True)
    y_ref[...] = dev * lax.rsqrt(var + 1e-5) * g_ref[...] + be_ref[...]


def _fin_call(ap, s, dp, b, g, be):
    RB = 1000
    return pl.pallas_call(
        _fin_body,
        grid=(N // RB,),
        in_specs=[
            pl.BlockSpec((1, RB, D), lambda i: (0, i, 0)),
            pl.BlockSpec((1, RB, D), lambda i: (1, i, 0)),
            pl.BlockSpec((RB, D), lambda i: (i, 0)),
            pl.BlockSpec((1, RB, D), lambda i: (0, i, 0)),
            pl.BlockSpec((1, RB, D), lambda i: (1, i, 0)),
            pl.BlockSpec((1, D), lambda i: (0, 0)),
            pl.BlockSpec((1, D), lambda i: (0, 0)),
            pl.BlockSpec((1, D), lambda i: (0, 0)),
        ],
        out_specs=pl.BlockSpec((RB, D), lambda i: (i, 0)),
        out_shape=jax.ShapeDtypeStruct((N, D), jnp.float32),
    )(ap, ap, s, dp, dp, b, g, be)


def kernel(xs, edge_index, W, b, ln_gamma, ln_beta):
    row = edge_index[0]
    col = edge_index[1]
    pad = EP - E
    row_p = jnp.concatenate([row, jnp.zeros((pad,), jnp.int32)])
    col_p = jnp.concatenate([col, jnp.full((pad,), N, jnp.int32)])
    rowi = row_p.reshape(NBLK, K)
    coli = col_p.reshape(NBLK, K)

    zeros128 = jnp.zeros((NPAD, D), jnp.float32)

    ones128 = jnp.ones((K, D), jnp.float32)
    dp = _deg_call(coli, ones128, zeros128)
    z, s = _prep_call(xs, W, dp)
    ap = _agg_call(z, rowi, coli, zeros128)
    b2 = b.reshape(1, D)
    g2 = ln_gamma.reshape(1, D)
    be2 = ln_beta.reshape(1, D)
    return _fin_call(ap, s, dp, b2, g2, be2)


# speedup vs baseline: 1.0191x; 1.0191x over previous
import functools

import jax
import jax.numpy as jnp
from jax import lax
from jax.experimental import pallas as pl
from jax.experimental.pallas import tpu as pltpu
from jax.experimental.pallas import tpu_sc as plsc

N = 10000
E = 320000
D = 128

NC = 2
NS = 16
NW = NC * NS
K = 128
BPW = 80
NBLK = BPW * NW
EP = NBLK * K
BPW0 = 112
BPW1 = 2 * BPW - BPW0
NPAD = 10240
SLAB = NPAD // NS


def _sc_mesh():
    return plsc.VectorSubcoreMesh(core_axis_name="c", subcore_axis_name="s")


def _deg_body(coli_hbm, ones_hbm, zeros_hbm, out, cidx_v, cidx_cur, ones_v,
              acc):
    cid = lax.axis_index("c")
    sid = lax.axis_index("s")
    wid = cid * NS + sid
    pltpu.sync_copy(coli_hbm.at[pl.ds(wid * BPW, BPW)], cidx_v)
    pltpu.sync_copy(ones_hbm, ones_v)
    pltpu.sync_copy(zeros_hbm.at[pl.ds(sid * SLAB, SLAB)],
                    acc.at[pl.ds(sid * SLAB, SLAB)])
    plsc.subcore_barrier()

    def step(i, carry):
        for j in range(K // 16):
            cidx_cur[pl.ds(j * 16, 16)] = cidx_v[i, pl.ds(j * 16, 16)]
        pltpu.sync_copy(ones_v, acc.at[cidx_cur], add=True)
        return carry

    lax.fori_loop(0, BPW, step, 0)
    plsc.subcore_barrier()
    pltpu.sync_copy(acc.at[pl.ds(sid * SLAB, SLAB)],
                    out.at[cid, pl.ds(sid * SLAB, SLAB)])


def _deg_call(coli, ones128, zeros128):
    f = functools.partial(
        pl.kernel,
        out_type=jax.ShapeDtypeStruct((NC, NPAD, D), jnp.float32),
        mesh=_sc_mesh(),
        scratch_types=[
            pltpu.VMEM((BPW, K), jnp.int32),
            pltpu.VMEM((K,), jnp.int32),
            pltpu.VMEM((K, D), jnp.float32),
            pltpu.VMEM_SHARED((NPAD, D), jnp.float32),
        ],
    )(_deg_body)
    return f(coli, ones128, zeros128)


NBUF = 4
H = K // 2
CH = 16


def _agg_body(z_hbm, rowi_hbm, coli_hbm, zeros_hbm, out,
              ridx_c, cidx_c, curs, rows, acc, gsems):
    cid = lax.axis_index("c")
    sid = lax.axis_index("s")
    base = jnp.where(cid == 0, sid * BPW0, NS * BPW0 + sid * BPW1)
    nchunks = jnp.where(cid == 0, BPW0 // CH, BPW1 // CH)
    pltpu.sync_copy(zeros_hbm.at[pl.ds(sid * SLAB, SLAB)],
                    acc.at[pl.ds(sid * SLAB, SLAB)])
    plsc.subcore_barrier()

    SB = 2 * CH

    def fire(s, b):
        pltpu.async_copy(
            z_hbm.at[ridx_c.at[s // 2, pl.ds((s % 2) * H, H)]],
            rows[b], gsems[b])

    def drain(b):
        pltpu.make_async_copy(z_hbm.at[ridx_c.at[0, pl.ds(0, H)]],
                              rows[b], gsems[b]).wait()

    def stage(s, cur):
        for j in range(H // 16):
            cur[pl.ds(j * 16, 16)] = cidx_c[s // 2,
                                            pl.ds((s % 2) * H + j * 16, 16)]

    def chunk(q, carry):
        pltpu.sync_copy(rowi_hbm.at[pl.ds(base + q * CH, CH)], ridx_c)
        pltpu.sync_copy(coli_hbm.at[pl.ds(base + q * CH, CH)], cidx_c)
        for s in range(NBUF - 1):
            fire(s, s)
        for s in range(SB):
            b = s % NBUF
            if s + NBUF - 1 < SB:
                fire(s + NBUF - 1, (s + NBUF - 1) % NBUF)
            stage(s, curs[b])
            drain(b)
            pltpu.sync_copy(rows[b], acc.at[curs[b]], add=True)
        return carry

    lax.fori_loop(0, nchunks, chunk, 0)
    plsc.subcore_barrier()
    pltpu.sync_copy(acc.at[pl.ds(sid * SLAB, SLAB)],
                    out.at[cid, pl.ds(sid * SLAB, SLAB)])


def _agg_call(z, rowi, coli, zeros128):
    f = functools.partial(
        pl.kernel,
        out_type=jax.ShapeDtypeStruct((NC, NPAD, D), jnp.float32),
        mesh=_sc_mesh(),
        scratch_types=[
            pltpu.VMEM((CH, K), jnp.int32),
            pltpu.VMEM((CH, K), jnp.int32),
            [pltpu.VMEM((H,), jnp.int32) for _ in range(NBUF)],
            [pltpu.VMEM((H, D), jnp.float32) for _ in range(NBUF)],
            pltpu.VMEM_SHARED((NPAD, D), jnp.float32),
            [pltpu.SemaphoreType.DMA for _ in range(NBUF)],
        ],
    )(_agg_body)
    return f(z, rowi, coli, zeros128)


def _prep_body(xs_ref, w_ref, d0_ref, d1_ref, z_ref, s_ref):
    deg = d0_ref[0, :, 0:1] + d1_ref[0, :, 0:1] + 1.0
    dinv = lax.rsqrt(deg)
    xw = lax.dot_general(xs_ref[...], w_ref[...], (((1,), (1,)), ((), ())),
                         preferred_element_type=jnp.float32)
    z = dinv * xw
    z_ref[...] = z
    s_ref[...] = dinv * z


def _prep_call(xs, W, dp):
    RB = 1000
    return pl.pallas_call(
        _prep_body,
        grid=(N // RB,),
        in_specs=[
            pl.BlockSpec((RB, D), lambda i: (i, 0)),
            pl.BlockSpec((D, D), lambda i: (0, 0)),
            pl.BlockSpec((1, RB, D), lambda i: (0, i, 0)),
            pl.BlockSpec((1, RB, D), lambda i: (1, i, 0)),
        ],
        out_specs=[
            pl.BlockSpec((RB, D), lambda i: (i, 0)),
            pl.BlockSpec((RB, D), lambda i: (i, 0)),
        ],
        out_shape=[
            jax.ShapeDtypeStruct((N, D), jnp.float32),
            jax.ShapeDtypeStruct((N, D), jnp.float32),
        ],
    )(xs, W, dp, dp)


def _fin_body(a0_ref, a1_ref, s_ref, d0_ref, d1_ref, b_ref, g_ref, be_ref,
              y_ref):
    deg = d0_ref[0, :, 0:1] + d1_ref[0, :, 0:1] + 1.0
    dinv = lax.rsqrt(deg)
    out = (dinv * (a0_ref[0] + a1_ref[0]) + s_ref[...] + b_ref[...])
    mu = jnp.mean(out, axis=1, keepdims=True)
    dev = out - mu
    var = jnp.mean(dev * dev, axis=1, keepdims=True)
    y_ref[...] = dev * lax.rsqrt(var + 1e-5) * g_ref[...] + be_ref[...]


def _fin_call(ap, s, dp, b, g, be):
    RB = 1000
    return pl.pallas_call(
        _fin_body,
        grid=(N // RB,),
        in_specs=[
            pl.BlockSpec((1, RB, D), lambda i: (0, i, 0)),
            pl.BlockSpec((1, RB, D), lambda i: (1, i, 0)),
            pl.BlockSpec((RB, D), lambda i: (i, 0)),
            pl.BlockSpec((1, RB, D), lambda i: (0, i, 0)),
            pl.BlockSpec((1, RB, D), lambda i: (1, i, 0)),
            pl.BlockSpec((1, D), lambda i: (0, 0)),
            pl.BlockSpec((1, D), lambda i: (0, 0)),
            pl.BlockSpec((1, D), lambda i: (0, 0)),
        ],
        out_specs=pl.BlockSpec((RB, D), lambda i: (i, 0)),
        out_shape=jax.ShapeDtypeStruct((N, D), jnp.float32),
    )(ap, ap, s, dp, dp, b, g, be)


def kernel(xs, edge_index, W, b, ln_gamma, ln_beta):
    row = edge_index[0]
    col = edge_index[1]
    pad = EP - E
    row_p = jnp.concatenate([row, jnp.zeros((pad,), jnp.int32)])
    col_p = jnp.concatenate([col, jnp.full((pad,), N, jnp.int32)])
    rowi = row_p.reshape(NBLK, K)
    coli = col_p.reshape(NBLK, K)

    zeros128 = jnp.zeros((NPAD, D), jnp.float32)

    ones128 = jnp.ones((K, D), jnp.float32)
    dp = _deg_call(coli, ones128, zeros128)
    z, s = _prep_call(xs, W, dp)
    ap = _agg_call(z, rowi, coli, zeros128)
    b2 = b.reshape(1, D)
    g2 = ln_gamma.reshape(1, D)
    be2 = ln_beta.reshape(1, D)
    return _fin_call(ap, s, dp, b2, g2, be2)


# speedup vs baseline: 1.0238x; 1.0046x over previous
import functools

import jax
import jax.numpy as jnp
from jax import lax
from jax.experimental import pallas as pl
from jax.experimental.pallas import tpu as pltpu
from jax.experimental.pallas import tpu_sc as plsc

N = 10000
E = 320000
D = 128

NC = 2
NS = 16
NW = NC * NS
K = 128
BPW = 80
NBLK = BPW * NW
EP = NBLK * K
BPW0 = 128
BPW1 = 2 * BPW - BPW0
NPAD = 10240
SLAB = NPAD // NS


def _sc_mesh():
    return plsc.VectorSubcoreMesh(core_axis_name="c", subcore_axis_name="s")


def _deg_body(coli_hbm, ones_hbm, zeros_hbm, out, cidx_v, cidx_cur, ones_v,
              acc):
    cid = lax.axis_index("c")
    sid = lax.axis_index("s")
    wid = cid * NS + sid
    pltpu.sync_copy(coli_hbm.at[pl.ds(wid * BPW, BPW)], cidx_v)
    pltpu.sync_copy(ones_hbm, ones_v)
    pltpu.sync_copy(zeros_hbm.at[pl.ds(sid * SLAB, SLAB)],
                    acc.at[pl.ds(sid * SLAB, SLAB)])
    plsc.subcore_barrier()

    def step(i, carry):
        for j in range(K // 16):
            cidx_cur[pl.ds(j * 16, 16)] = cidx_v[i, pl.ds(j * 16, 16)]
        pltpu.sync_copy(ones_v, acc.at[cidx_cur], add=True)
        return carry

    lax.fori_loop(0, BPW, step, 0)
    plsc.subcore_barrier()
    pltpu.sync_copy(acc.at[pl.ds(sid * SLAB, SLAB)],
                    out.at[cid, pl.ds(sid * SLAB, SLAB)])


def _deg_call(coli, ones128, zeros128):
    f = functools.partial(
        pl.kernel,
        out_type=jax.ShapeDtypeStruct((NC, NPAD, D), jnp.float32),
        mesh=_sc_mesh(),
        scratch_types=[
            pltpu.VMEM((BPW, K), jnp.int32),
            pltpu.VMEM((K,), jnp.int32),
            pltpu.VMEM((K, D), jnp.float32),
            pltpu.VMEM_SHARED((NPAD, D), jnp.float32),
        ],
    )(_deg_body)
    return f(coli, ones128, zeros128)


NBUF = 4
H = K // 2
CH = 16


def _agg_body(z_hbm, rowi_hbm, coli_hbm, zeros_hbm, out,
              ridx_c, cidx_c, curs, rows, acc, gsems):
    cid = lax.axis_index("c")
    sid = lax.axis_index("s")
    base = jnp.where(cid == 0, sid * BPW0, NS * BPW0 + sid * BPW1)
    nchunks = jnp.where(cid == 0, BPW0 // CH, BPW1 // CH)
    pltpu.sync_copy(zeros_hbm.at[pl.ds(sid * SLAB, SLAB)],
                    acc.at[pl.ds(sid * SLAB, SLAB)])
    plsc.subcore_barrier()

    SB = 2 * CH

    def fire(s, b):
        pltpu.async_copy(
            z_hbm.at[ridx_c.at[s // 2, pl.ds((s % 2) * H, H)]],
            rows[b], gsems[b])

    def drain(b):
        pltpu.make_async_copy(z_hbm.at[ridx_c.at[0, pl.ds(0, H)]],
                              rows[b], gsems[b]).wait()

    def stage(s, cur):
        for j in range(H // 16):
            cur[pl.ds(j * 16, 16)] = cidx_c[s // 2,
                                            pl.ds((s % 2) * H + j * 16, 16)]

    def chunk(q, carry):
        pltpu.sync_copy(rowi_hbm.at[pl.ds(base + q * CH, CH)], ridx_c)
        pltpu.sync_copy(coli_hbm.at[pl.ds(base + q * CH, CH)], cidx_c)
        for s in range(NBUF - 1):
            fire(s, s)
        for s in range(SB):
            b = s % NBUF
            if s + NBUF - 1 < SB:
                fire(s + NBUF - 1, (s + NBUF - 1) % NBUF)
            stage(s, curs[b])
            drain(b)
            pltpu.sync_copy(rows[b], acc.at[curs[b]], add=True)
        return carry

    lax.fori_loop(0, nchunks, chunk, 0)
    plsc.subcore_barrier()
    pltpu.sync_copy(acc.at[pl.ds(sid * SLAB, SLAB)],
                    out.at[cid, pl.ds(sid * SLAB, SLAB)])


def _agg_call(z, rowi, coli, zeros128):
    f = functools.partial(
        pl.kernel,
        out_type=jax.ShapeDtypeStruct((NC, NPAD, D), jnp.float32),
        mesh=_sc_mesh(),
        scratch_types=[
            pltpu.VMEM((CH, K), jnp.int32),
            pltpu.VMEM((CH, K), jnp.int32),
            [pltpu.VMEM((H,), jnp.int32) for _ in range(NBUF)],
            [pltpu.VMEM((H, D), jnp.float32) for _ in range(NBUF)],
            pltpu.VMEM_SHARED((NPAD, D), jnp.float32),
            [pltpu.SemaphoreType.DMA for _ in range(NBUF)],
        ],
    )(_agg_body)
    return f(z, rowi, coli, zeros128)


def _prep_body(xs_ref, w_ref, d0_ref, d1_ref, z_ref, s_ref):
    deg = d0_ref[0, :, 0:1] + d1_ref[0, :, 0:1] + 1.0
    dinv = lax.rsqrt(deg)
    xw = lax.dot_general(xs_ref[...], w_ref[...], (((1,), (1,)), ((), ())),
                         preferred_element_type=jnp.float32)
    z = dinv * xw
    z_ref[...] = z
    s_ref[...] = dinv * z


def _prep_call(xs, W, dp):
    RB = 1000
    return pl.pallas_call(
        _prep_body,
        grid=(N // RB,),
        in_specs=[
            pl.BlockSpec((RB, D), lambda i: (i, 0)),
            pl.BlockSpec((D, D), lambda i: (0, 0)),
            pl.BlockSpec((1, RB, D), lambda i: (0, i, 0)),
            pl.BlockSpec((1, RB, D), lambda i: (1, i, 0)),
        ],
        out_specs=[
            pl.BlockSpec((RB, D), lambda i: (i, 0)),
            pl.BlockSpec((RB, D), lambda i: (i, 0)),
        ],
        out_shape=[
            jax.ShapeDtypeStruct((N, D), jnp.float32),
            jax.ShapeDtypeStruct((N, D), jnp.float32),
        ],
    )(xs, W, dp, dp)


def _fin_body(a0_ref, a1_ref, s_ref, d0_ref, d1_ref, b_ref, g_ref, be_ref,
              y_ref):
    deg = d0_ref[0, :, 0:1] + d1_ref[0, :, 0:1] + 1.0
    dinv = lax.rsqrt(deg)
    out = (dinv * (a0_ref[0] + a1_ref[0]) + s_ref[...] + b_ref[...])
    mu = jnp.mean(out, axis=1, keepdims=True)
    dev = out - mu
    var = jnp.mean(dev * dev, axis=1, keepdims=True)
    y_ref[...] = dev * lax.rsqrt(var + 1e-5) * g_ref[...] + be_ref[...]


def _fin_call(ap, s, dp, b, g, be):
    RB = 1000
    return pl.pallas_call(
        _fin_body,
        grid=(N // RB,),
        in_specs=[
            pl.BlockSpec((1, RB, D), lambda i: (0, i, 0)),
            pl.BlockSpec((1, RB, D), lambda i: (1, i, 0)),
            pl.BlockSpec((RB, D), lambda i: (i, 0)),
            pl.BlockSpec((1, RB, D), lambda i: (0, i, 0)),
            pl.BlockSpec((1, RB, D), lambda i: (1, i, 0)),
            pl.BlockSpec((1, D), lambda i: (0, 0)),
            pl.BlockSpec((1, D), lambda i: (0, 0)),
            pl.BlockSpec((1, D), lambda i: (0, 0)),
        ],
        out_specs=pl.BlockSpec((RB, D), lambda i: (i, 0)),
        out_shape=jax.ShapeDtypeStruct((N, D), jnp.float32),
    )(ap, ap, s, dp, dp, b, g, be)


def kernel(xs, edge_index, W, b, ln_gamma, ln_beta):
    row = edge_index[0]
    col = edge_index[1]
    pad = EP - E
    row_p = jnp.concatenate([row, jnp.zeros((pad,), jnp.int32)])
    col_p = jnp.concatenate([col, jnp.full((pad,), N, jnp.int32)])
    rowi = row_p.reshape(NBLK, K)
    coli = col_p.reshape(NBLK, K)

    zeros128 = jnp.zeros((NPAD, D), jnp.float32)

    ones128 = jnp.ones((K, D), jnp.float32)
    dp = _deg_call(coli, ones128, zeros128)
    z, s = _prep_call(xs, W, dp)
    ap = _agg_call(z, rowi, coli, zeros128)
    b2 = b.reshape(1, D)
    g2 = ln_gamma.reshape(1, D)
    be2 = ln_beta.reshape(1, D)
    return _fin_call(ap, s, dp, b2, g2, be2)
